# baseline (speedup 1.0000x reference)
import jax
import jax.numpy as jnp
from jax.experimental import pallas as pl

N = 10000
D2 = 64


def kernel(x, edge_index, edge_attr, W1_rel, b1, W1_root, W2_rel, b2, W2_root, fc_W, fc_b, fc2_W, fc2_b, dec_W1, dec_b1, dec_W2, dec_b2):
    src = edge_index[0]
    dst = edge_index[1]
    msg = x[src] * edge_attr[:, None]
    agg = jax.ops.segment_sum(msg, dst, num_segments=N)
    h = agg @ W1_rel.T + b1 + x @ W1_root.T
    h = jax.nn.relu(h)
    msg2 = h[src] * edge_attr[:, None]
    agg2 = jax.ops.segment_sum(msg2, dst, num_segments=N)
    z = agg2 @ W2_rel.T + b2 + h @ W2_root.T
    z = z.reshape(-1, N * D2)
    z = jax.nn.relu(z)
    z = z @ fc_W.T + fc_b
    zz = z @ fc2_W.T + fc2_b
    zz = jax.nn.relu(zz)
    zz = zz.reshape(-1, D2)
    e = jnp.concatenate([zz[src], zz[dst]], axis=1)
    e = jax.nn.relu(e @ dec_W1.T + dec_b1)
    e = e @ dec_W2.T + dec_b2
    return jax.nn.sigmoid(e).squeeze(-1)


# speedup vs baseline: 1.4746x; 1.4746x over previous
import functools

import jax
import jax.numpy as jnp
from jax import lax
from jax.experimental import pallas as pl
from jax.experimental.pallas import tpu as pltpu
from jax.experimental.pallas import tpu_sc as plsc

NN = 10000
NP = 10240
DIN = 128
DD1 = 128
DD2 = 64
EP = 163840
CH = 128
NW = 32
CPW = EP // (NW * CH)
ROWS_PER_SUB = NP // 16
_F32 = jnp.float32
_DN = (((1,), (1,)), ((), ()))


def _zero_chunks():
    out, off = [], 0
    while off < ROWS_PER_SUB:
        sz = min(CH, ROWS_PER_SUB - off)
        out.append((off, sz))
        off += sz
    return out


def _make_sc_agg(SW):
    D = 128
    mesh = plsc.VectorSubcoreMesh(core_axis_name="c", subcore_axis_name="s")

    @functools.partial(
        pl.kernel,
        out_type=jax.ShapeDtypeStruct((2, NP, D), _F32),
        mesh=mesh,
        scratch_types=[
            pltpu.VMEM((CPW, CH), jnp.int32),
            pltpu.VMEM((CPW, CH), jnp.int32),
            pltpu.VMEM((CPW * CH,), _F32),
            pltpu.VMEM((CH, D), _F32),
            pltpu.VMEM_SHARED((NP, D), _F32),
            pltpu.SemaphoreType.DMA,
        ],
    )
    def k(feat, src2, dst2, ea1, out, src_v, dst_v, ea_v, rows_v, acc, sem):
        c = lax.axis_index("c")
        s = lax.axis_index("s")
        wid = s * 2 + c

        def zrow(i, _):
            for kk in range(D // 16):
                rows_v[i, pl.ds(16 * kk, 16)] = jnp.zeros((16,), _F32)
            return 0

        lax.fori_loop(0, CH, zrow, 0)
        for off, sz in _zero_chunks():
            base = pl.multiple_of(s * ROWS_PER_SUB + off, CH)
            pltpu.sync_copy(rows_v.at[pl.ds(0, sz)], acc.at[pl.ds(base, sz)])

        pltpu.sync_copy(src2.at[pl.ds(wid * CPW, CPW)], src_v)
        pltpu.sync_copy(dst2.at[pl.ds(wid * CPW, CPW)], dst_v)
        pltpu.sync_copy(ea1.at[pl.ds(wid * CPW * CH, CPW * CH)], ea_v)
        plsc.subcore_barrier()

        def chunk(j, _):
            pltpu.async_copy(feat.at[src_v.at[j]], rows_v, sem).wait()

            def scale(t, _):
                gbase = pl.multiple_of(j * CH + t * 16, 16)
                eav16 = ea_v[pl.ds(gbase, 16)]
                for i in range(16):
                    row = t * 16 + i
                    splat = jnp.full((16,), eav16[i])
                    for kk in range(SW // 16):
                        sl = pl.ds(16 * kk, 16)
                        rows_v[row, sl] = rows_v[row, sl] * splat
                return 0

            lax.fori_loop(0, CH // 16, scale, 0)
            pltpu.sync_copy(rows_v, acc.at[dst_v.at[j]], add=True)
            return 0

        lax.fori_loop(0, CPW, chunk, 0)
        plsc.subcore_barrier()
        for off, sz in _zero_chunks():
            base = pl.multiple_of(s * ROWS_PER_SUB + off, CH)
            pltpu.sync_copy(acc.at[pl.ds(base, sz)], out.at[c, pl.ds(base, sz)])

    return k


_sc_agg128 = _make_sc_agg(DIN)
_sc_agg64 = _make_sc_agg(DD2)


def _make_sc_dec():
    mesh = plsc.VectorSubcoreMesh(core_axis_name="c", subcore_axis_name="s")

    @functools.partial(
        pl.kernel,
        out_type=jax.ShapeDtypeStruct((EP, 2 * DD2), _F32),
        mesh=mesh,
        scratch_types=[
            pltpu.VMEM((CPW, CH), jnp.int32),
            pltpu.VMEM((CPW, CH), jnp.int32),
            pltpu.VMEM((CH, 2 * DD2), _F32),
            pltpu.SemaphoreType.DMA,
        ],
    )
    def k(av, bv, src2, dst2, g, src_v, dst_v, buf, sem):
        c = lax.axis_index("c")
        s = lax.axis_index("s")
        wid = s * 2 + c
        pltpu.sync_copy(src2.at[pl.ds(wid * CPW, CPW)], src_v)
        pltpu.sync_copy(dst2.at[pl.ds(wid * CPW, CPW)], dst_v)

        def chunk(j, _):
            pltpu.async_copy(av.at[src_v.at[j]], buf, sem).wait()
            pltpu.async_copy(bv.at[dst_v.at[j]], buf, sem, add=True).wait()
            pltpu.sync_copy(buf, g.at[pl.ds(wid * (CPW * CH) + j * CH, CH)])
            return 0

        lax.fori_loop(0, CPW, chunk, 0)

    return k


_sc_dec = _make_sc_dec()


def _tc1_body(aggp, xr, w1r, w1o, b1r, w2r, w2o, b2r, hw, hroot):
    agg = aggp[0] + aggp[1]
    h = lax.dot_general(agg, w1r[...], _DN, preferred_element_type=_F32)
    h = h + lax.dot_general(xr[...], w1o[...], _DN, preferred_element_type=_F32)
    h = jnp.maximum(h + b1r[...][None, :], 0.0)
    hwv = lax.dot_general(h, w2r[...], _DN, preferred_element_type=_F32)
    hw[...] = jnp.concatenate(
        [hwv, jnp.zeros((hwv.shape[0], DIN - DD2), _F32)], axis=1)
    hroot[...] = (lax.dot_general(h, w2o[...], _DN, preferred_element_type=_F32)
                  + b2r[...][None, :])


_TC1_BN = 1000


def _tc1(aggp, x, w1r, w1o, b1, w2r, w2o, b2):
    grid = (NN // _TC1_BN,)
    return pl.pallas_call(
        _tc1_body,
        grid=grid,
        in_specs=[
            pl.BlockSpec((2, _TC1_BN, DIN), lambda i: (0, i, 0)),
            pl.BlockSpec((_TC1_BN, DIN), lambda i: (i, 0)),
            pl.BlockSpec((DD1, DIN), lambda i: (0, 0)),
            pl.BlockSpec((DD1, DIN), lambda i: (0, 0)),
            pl.BlockSpec((DD1,), lambda i: (0,)),
            pl.BlockSpec((DD2, DD1), lambda i: (0, 0)),
            pl.BlockSpec((DD2, DD1), lambda i: (0, 0)),
            pl.BlockSpec((DD2,), lambda i: (0,)),
        ],
        out_specs=[
            pl.BlockSpec((_TC1_BN, DIN), lambda i: (i, 0)),
            pl.BlockSpec((_TC1_BN, DD2), lambda i: (i, 0)),
        ],
        out_shape=[
            jax.ShapeDtypeStruct((NN, DIN), _F32),
            jax.ShapeDtypeStruct((NN, DD2), _F32),
        ],
    )(aggp, x, w1r, w1o, b1, w2r, w2o, b2)


def _tc2_body(aggp, hroot, fcw, fcb, z2):
    zr = jnp.maximum(aggp[0] + aggp[1] + hroot[...], 0.0)
    p = jnp.sum(fcw[...] * zr[None, :], axis=1)

    @pl.when(pl.program_id(0) == 0)
    def _():
        z2[...] = p + fcb[...]

    @pl.when(pl.program_id(0) != 0)
    def _():
        z2[...] = z2[...] + p


_TC2_BC = 25600


def _tc2(aggp_f, hroot_f, fc_W, fc_b):
    grid = (NN * DD2 // _TC2_BC,)
    return pl.pallas_call(
        _tc2_body,
        grid=grid,
        in_specs=[
            pl.BlockSpec((2, _TC2_BC), lambda i: (0, i)),
            pl.BlockSpec((_TC2_BC,), lambda i: (i,)),
            pl.BlockSpec((DD2, _TC2_BC), lambda i: (0, i)),
            pl.BlockSpec((DD2,), lambda i: (0,)),
        ],
        out_specs=pl.BlockSpec((DD2,), lambda i: (0,)),
        out_shape=jax.ShapeDtypeStruct((DD2,), _F32),
    )(aggp_f, hroot_f, fc_W, fc_b)


def _tc3_body(w4, fcb2, z2, dwa, dwb, db1, av, bv):
    t = jnp.sum(w4[...] * z2[...][None, None, :], axis=2) + fcb2[...]
    zz = jnp.maximum(t, 0.0)
    av[...] = (lax.dot_general(zz, dwa[...], _DN, preferred_element_type=_F32)
               + db1[...][None, :])
    bv[...] = lax.dot_general(zz, dwb[...], _DN, preferred_element_type=_F32)


_TC3_BN = 400


def _tc3(w4, fcb2, z2, dwa, dwb, db1):
    grid = (NN // _TC3_BN,)
    return pl.pallas_call(
        _tc3_body,
        grid=grid,
        in_specs=[
            pl.BlockSpec((_TC3_BN, DD2, DD2), lambda i: (i, 0, 0)),
            pl.BlockSpec((_TC3_BN, DD2), lambda i: (i, 0)),
            pl.BlockSpec((DD2,), lambda i: (0,)),
            pl.BlockSpec((DD1, DD2), lambda i: (0, 0)),
            pl.BlockSpec((DD1, DD2), lambda i: (0, 0)),
            pl.BlockSpec((DD1,), lambda i: (0,)),
        ],
        out_specs=[
            pl.BlockSpec((_TC3_BN, DD1), lambda i: (i, 0)),
            pl.BlockSpec((_TC3_BN, DD1), lambda i: (i, 0)),
        ],
        out_shape=[
            jax.ShapeDtypeStruct((NN, DD1), _F32),
            jax.ShapeDtypeStruct((NN, DD1), _F32),
        ],
    )(w4, fcb2, z2, dwa, dwb, db1)


def _tc4_body(g, w2, b2, out):
    t = jnp.sum(jnp.maximum(g[...], 0.0) * w2[...][0][None, :], axis=1) + b2[0]
    out[...] = 1.0 / (1.0 + jnp.exp(-t))


_TC4_BE = 2048


def _tc4(g, dec_W2, dec_b2):
    grid = (EP // _TC4_BE,)
    return pl.pallas_call(
        _tc4_body,
        grid=grid,
        in_specs=[
            pl.BlockSpec((_TC4_BE, DD1), lambda i: (i, 0)),
            pl.BlockSpec((1, DD1), lambda i: (0, 0)),
            pl.BlockSpec(memory_space=pltpu.SMEM),
        ],
        out_specs=pl.BlockSpec((_TC4_BE,), lambda i: (i,)),
        out_shape=jax.ShapeDtypeStruct((EP,), _F32),
    )(g, dec_W2, dec_b2)


def kernel(x, edge_index, edge_attr, W1_rel, b1, W1_root, W2_rel, b2, W2_root,
           fc_W, fc_b, fc2_W, fc2_b, dec_W1, dec_b1, dec_W2, dec_b2):
    E = edge_index.shape[1]
    pad = EP - E
    src_p = jnp.concatenate(
        [edge_index[0], jnp.zeros((pad,), jnp.int32)]).reshape(EP // CH, CH)
    dst_p = jnp.concatenate(
        [edge_index[1], jnp.zeros((pad,), jnp.int32)]).reshape(EP // CH, CH)
    ea_p = jnp.concatenate([edge_attr, jnp.zeros((pad,), _F32)])

    aggp = _sc_agg128(x, src_p, dst_p, ea_p)
    hw, hroot = _tc1(aggp, x, W1_rel, W1_root, b1, W2_rel, W2_root, b2)
    agg2p = _sc_agg64(hw, src_p, dst_p, ea_p)
    a2f = agg2p[:, :NN, :DD2].reshape(2, NN * DD2)
    z2 = _tc2(a2f, hroot.reshape(NN * DD2), fc_W, fc_b)
    av, bv = _tc3(fc2_W.reshape(NN, DD2, DD2), fc2_b.reshape(NN, DD2), z2,
                  dec_W1[:, :DD2], dec_W1[:, DD2:], dec_b1)
    g = _sc_dec(av, bv, src_p, dst_p)
    outp = _tc4(g, dec_W2, dec_b2)
    return outp[:E]


# speedup vs baseline: 1.5940x; 1.0810x over previous
import functools

import jax
import jax.numpy as jnp
from jax import lax
from jax.experimental import pallas as pl
from jax.experimental.pallas import tpu as pltpu
from jax.experimental.pallas import tpu_sc as plsc

NN = 10000
NP = 10240
DIN = 128
DD1 = 128
DD2 = 64
EP = 163840
CH = 128
NW = 32
CPW = EP // (NW * CH)
ROWS_PER_SUB = NP // 16
NBUF_A = 2
NBUF_D = 4
_F32 = jnp.float32
_DN = (((1,), (1,)), ((), ()))


def _zero_chunks():
    out, off = [], 0
    while off < ROWS_PER_SUB:
        sz = min(CH, ROWS_PER_SUB - off)
        out.append((off, sz))
        off += sz
    return out


def _make_sc_agg(SW):
    D = 128
    mesh = plsc.VectorSubcoreMesh(core_axis_name="c", subcore_axis_name="s")

    @functools.partial(
        pl.kernel,
        out_type=jax.ShapeDtypeStruct((2, NP, D), _F32),
        mesh=mesh,
        scratch_types=[
            pltpu.VMEM((CPW, CH), jnp.int32),
            pltpu.VMEM((CPW, CH), jnp.int32),
            pltpu.VMEM((CPW * CH,), _F32),
        ] + [pltpu.VMEM((CH, D), _F32)] * NBUF_A
          + [pltpu.VMEM_SHARED((NP, D), _F32)]
          + [pltpu.SemaphoreType.DMA] * NBUF_A,
    )
    def k(feat, src2, dst2, ea1, out, src_v, dst_v, ea_v,
          rb0, rb1, acc, sm0, sm1):
        bufs = [rb0, rb1]
        sems = [sm0, sm1]
        c = lax.axis_index("c")
        s = lax.axis_index("s")
        wid = s * 2 + c

        def zrow(i, _):
            for kk in range(D // 16):
                rb0[i, pl.ds(16 * kk, 16)] = jnp.zeros((16,), _F32)
            return 0

        lax.fori_loop(0, CH, zrow, 0)
        for off, sz in _zero_chunks():
            base = pl.multiple_of(s * ROWS_PER_SUB + off, CH)
            pltpu.sync_copy(rb0.at[pl.ds(0, sz)], acc.at[pl.ds(base, sz)])

        pltpu.sync_copy(src2.at[pl.ds(wid * CPW, CPW)], src_v)
        pltpu.sync_copy(dst2.at[pl.ds(wid * CPW, CPW)], dst_v)
        pltpu.sync_copy(ea1.at[pl.ds(wid * CPW * CH, CPW * CH)], ea_v)
        plsc.subcore_barrier()

        def wait(b):
            pltpu.make_async_copy(
                feat.at[pl.ds(0, CH)], bufs[b], sems[b]).wait()

        def pipe(t, _):
            for b in range(NBUF_A):
                j = t + b
                @pl.when(j < CPW)
                def _():
                    @pl.when(j >= NBUF_A)
                    def _():
                        wait(b)
                    pltpu.async_copy(feat.at[src_v.at[j]], bufs[b], sems[b])

                jj = j - 1
                bb = (b - 1) % NBUF_A

                @pl.when((jj >= 0) & (jj < CPW))
                def _():
                    wait(bb)

                    def scale(gg, _):
                        gbase = pl.multiple_of(jj * CH + gg * 16, 16)
                        eav16 = ea_v[pl.ds(gbase, 16)]
                        for i in range(16):
                            row = gg * 16 + i
                            splat = jnp.full((16,), eav16[i])
                            for kk in range(SW // 16):
                                sl = pl.ds(16 * kk, 16)
                                bufs[bb][row, sl] = bufs[bb][row, sl] * splat
                        return 0

                    lax.fori_loop(0, CH // 16, scale, 0)
                    pltpu.async_copy(bufs[bb], acc.at[dst_v.at[jj]], sems[bb],
                                     add=True)
            return 0

        lax.fori_loop(0, (CPW + NBUF_A) // NBUF_A,
                      lambda t, u: pipe(t * NBUF_A, u), 0, unroll=False)
        for b in range(NBUF_A):
            wait(b)
        plsc.subcore_barrier()
        for off, sz in _zero_chunks():
            base = pl.multiple_of(s * ROWS_PER_SUB + off, CH)
            pltpu.sync_copy(acc.at[pl.ds(base, sz)], out.at[c, pl.ds(base, sz)])

    return k


_sc_agg128 = _make_sc_agg(DIN)
_sc_agg64 = _make_sc_agg(DD2)


def _make_sc_dec():
    mesh = plsc.VectorSubcoreMesh(core_axis_name="c", subcore_axis_name="s")

    @functools.partial(
        pl.kernel,
        out_type=jax.ShapeDtypeStruct((EP, 2 * DD2), _F32),
        mesh=mesh,
        scratch_types=[
            pltpu.VMEM((CPW, CH), jnp.int32),
            pltpu.VMEM((CPW, CH), jnp.int32),
        ] + [pltpu.VMEM((CH, 2 * DD2), _F32)] * NBUF_D
          + [pltpu.SemaphoreType.DMA] * NBUF_D,
    )
    def k(av, bv, src2, dst2, g, src_v, dst_v,
          rb0, rb1, rb2, rb3, sm0, sm1, sm2, sm3):
        bufs = [rb0, rb1, rb2, rb3]
        sems = [sm0, sm1, sm2, sm3]
        c = lax.axis_index("c")
        s = lax.axis_index("s")
        wid = s * 2 + c
        pltpu.sync_copy(src2.at[pl.ds(wid * CPW, CPW)], src_v)
        pltpu.sync_copy(dst2.at[pl.ds(wid * CPW, CPW)], dst_v)

        def wait(b):
            pltpu.make_async_copy(
                av.at[pl.ds(0, CH)], bufs[b], sems[b]).wait()

        def pipe(t, _):
            for b in range(NBUF_D):
                j = t + b
                @pl.when(j < CPW)
                def _():
                    @pl.when(j >= NBUF_D)
                    def _():
                        wait(b)
                    pltpu.async_copy(av.at[src_v.at[j]], bufs[b], sems[b])

                j1 = j - 1
                bb1 = (b - 1) % NBUF_D

                @pl.when((j1 >= 0) & (j1 < CPW))
                def _():
                    wait(bb1)
                    pltpu.async_copy(bv.at[dst_v.at[j1]], bufs[bb1], sems[bb1],
                                     add=True)

                j2 = j - 2
                bb2 = (b - 2) % NBUF_D

                @pl.when((j2 >= 0) & (j2 < CPW))
                def _():
                    wait(bb2)
                    pltpu.async_copy(
                        bufs[bb2], g.at[pl.ds(wid * (CPW * CH) + j2 * CH, CH)],
                        sems[bb2])
            return 0

        lax.fori_loop(0, (CPW + NBUF_D) // NBUF_D,
                      lambda t, u: pipe(t * NBUF_D, u), 0, unroll=False)
        for b in range(NBUF_D):
            wait(b)

    return k


_sc_dec = _make_sc_dec()


def _tc1_body(aggp, xr, w1r, w1o, b1r, w2r, w2o, b2r, hw, hroot):
    agg = aggp[0] + aggp[1]
    h = lax.dot_general(agg, w1r[...], _DN, preferred_element_type=_F32)
    h = h + lax.dot_general(xr[...], w1o[...], _DN, preferred_element_type=_F32)
    h = jnp.maximum(h + b1r[...][None, :], 0.0)
    hwv = lax.dot_general(h, w2r[...], _DN, preferred_element_type=_F32)
    hw[...] = jnp.concatenate(
        [hwv, jnp.zeros((hwv.shape[0], DIN - DD2), _F32)], axis=1)
    hroot[...] = (lax.dot_general(h, w2o[...], _DN, preferred_element_type=_F32)
                  + b2r[...][None, :])


_TC1_BN = 1000


def _tc1(aggp, x, w1r, w1o, b1, w2r, w2o, b2):
    grid = (NN // _TC1_BN,)
    return pl.pallas_call(
        _tc1_body,
        grid=grid,
        in_specs=[
            pl.BlockSpec((2, _TC1_BN, DIN), lambda i: (0, i, 0)),
            pl.BlockSpec((_TC1_BN, DIN), lambda i: (i, 0)),
            pl.BlockSpec((DD1, DIN), lambda i: (0, 0)),
            pl.BlockSpec((DD1, DIN), lambda i: (0, 0)),
            pl.BlockSpec((DD1,), lambda i: (0,)),
            pl.BlockSpec((DD2, DD1), lambda i: (0, 0)),
            pl.BlockSpec((DD2, DD1), lambda i: (0, 0)),
            pl.BlockSpec((DD2,), lambda i: (0,)),
        ],
        out_specs=[
            pl.BlockSpec((_TC1_BN, DIN), lambda i: (i, 0)),
            pl.BlockSpec((_TC1_BN, DD2), lambda i: (i, 0)),
        ],
        out_shape=[
            jax.ShapeDtypeStruct((NN, DIN), _F32),
            jax.ShapeDtypeStruct((NN, DD2), _F32),
        ],
    )(aggp, x, w1r, w1o, b1, w2r, w2o, b2)


def _tc2_body(aggp, hroot, fcw, fcb, z2):
    zr = jnp.maximum(aggp[0] + aggp[1] + hroot[...], 0.0)
    p = jnp.sum(fcw[...] * zr[None, :], axis=1)

    @pl.when(pl.program_id(0) == 0)
    def _():
        z2[...] = p + fcb[...]

    @pl.when(pl.program_id(0) != 0)
    def _():
        z2[...] = z2[...] + p


_TC2_BC = 25600


def _tc2(aggp_f, hroot_f, fc_W, fc_b):
    grid = (NN * DD2 // _TC2_BC,)
    return pl.pallas_call(
        _tc2_body,
        grid=grid,
        in_specs=[
            pl.BlockSpec((2, _TC2_BC), lambda i: (0, i)),
            pl.BlockSpec((_TC2_BC,), lambda i: (i,)),
            pl.BlockSpec((DD2, _TC2_BC), lambda i: (0, i)),
            pl.BlockSpec((DD2,), lambda i: (0,)),
        ],
        out_specs=pl.BlockSpec((DD2,), lambda i: (0,)),
        out_shape=jax.ShapeDtypeStruct((DD2,), _F32),
    )(aggp_f, hroot_f, fc_W, fc_b)


def _tc3a_body(w2d, fcb2, z2, zzf):
    t = jnp.sum(w2d[...] * z2[...][None, :], axis=1) + fcb2[...]
    zzf[...] = jnp.maximum(t, 0.0)


_TC3A_BR = 5120


def _tc3a(fc2_W, fc2_b, z2):
    grid = (NN * DD2 // _TC3A_BR,)
    return pl.pallas_call(
        _tc3a_body,
        grid=grid,
        in_specs=[
            pl.BlockSpec((_TC3A_BR, DD2), lambda i: (i, 0)),
            pl.BlockSpec((_TC3A_BR,), lambda i: (i,)),
            pl.BlockSpec((DD2,), lambda i: (0,)),
        ],
        out_specs=pl.BlockSpec((_TC3A_BR,), lambda i: (i,)),
        out_shape=jax.ShapeDtypeStruct((NN * DD2,), _F32),
    )(fc2_W, fc2_b, z2)


def _tc3b_body(zz, dwa, dwb, db1, av, bv):
    z = zz[...]
    av[...] = (lax.dot_general(z, dwa[...], _DN, preferred_element_type=_F32)
               + db1[...][None, :])
    bv[...] = lax.dot_general(z, dwb[...], _DN, preferred_element_type=_F32)


_TC3B_BN = 1000


def _tc3b(zz2d, dwa, dwb, db1):
    grid = (NN // _TC3B_BN,)
    return pl.pallas_call(
        _tc3b_body,
        grid=grid,
        in_specs=[
            pl.BlockSpec((_TC3B_BN, DD2), lambda i: (i, 0)),
            pl.BlockSpec((DD1, DD2), lambda i: (0, 0)),
            pl.BlockSpec((DD1, DD2), lambda i: (0, 0)),
            pl.BlockSpec((DD1,), lambda i: (0,)),
        ],
        out_specs=[
            pl.BlockSpec((_TC3B_BN, DD1), lambda i: (i, 0)),
            pl.BlockSpec((_TC3B_BN, DD1), lambda i: (i, 0)),
        ],
        out_shape=[
            jax.ShapeDtypeStruct((NN, DD1), _F32),
            jax.ShapeDtypeStruct((NN, DD1), _F32),
        ],
    )(zz2d, dwa, dwb, db1)


def _tc4_body(g, w2, b2, out):
    t = jnp.sum(jnp.maximum(g[...], 0.0) * w2[...][0][None, :], axis=1) + b2[0]
    out[...] = 1.0 / (1.0 + jnp.exp(-t))


_TC4_BE = 2048


def _tc4(g, dec_W2, dec_b2):
    grid = (EP // _TC4_BE,)
    return pl.pallas_call(
        _tc4_body,
        grid=grid,
        in_specs=[
            pl.BlockSpec((_TC4_BE, DD1), lambda i: (i, 0)),
            pl.BlockSpec((1, DD1), lambda i: (0, 0)),
            pl.BlockSpec(memory_space=pltpu.SMEM),
        ],
        out_specs=pl.BlockSpec((_TC4_BE,), lambda i: (i,)),
        out_shape=jax.ShapeDtypeStruct((EP,), _F32),
    )(g, dec_W2, dec_b2)


def kernel(x, edge_index, edge_attr, W1_rel, b1, W1_root, W2_rel, b2, W2_root,
           fc_W, fc_b, fc2_W, fc2_b, dec_W1, dec_b1, dec_W2, dec_b2):
    E = edge_index.shape[1]
    pad = EP - E
    src_p = jnp.concatenate(
        [edge_index[0], jnp.zeros((pad,), jnp.int32)]).reshape(EP // CH, CH)
    dst_p = jnp.concatenate(
        [edge_index[1], jnp.zeros((pad,), jnp.int32)]).reshape(EP // CH, CH)
    ea_p = jnp.concatenate([edge_attr, jnp.zeros((pad,), _F32)])

    aggp = _sc_agg128(x, src_p, dst_p, ea_p)
    hw, hroot = _tc1(aggp, x, W1_rel, W1_root, b1, W2_rel, W2_root, b2)
    agg2p = _sc_agg64(hw, src_p, dst_p, ea_p)
    a2f = agg2p[:, :NN, :DD2].reshape(2, NN * DD2)
    z2 = _tc2(a2f, hroot.reshape(NN * DD2), fc_W, fc_b)
    zzf = _tc3a(fc2_W, fc2_b, z2)
    av, bv = _tc3b(zzf.reshape(NN, DD2), dec_W1[:, :DD2], dec_W1[:, DD2:],
                   dec_b1)
    g = _sc_dec(av, bv, src_p, dst_p)
    outp = _tc4(g, dec_W2, dec_b2)
    return outp[:E]


# speedup vs baseline: 1.7120x; 1.0740x over previous
import functools

import jax
import jax.numpy as jnp
from jax import lax
from jax.experimental import pallas as pl
from jax.experimental.pallas import tpu as pltpu
from jax.experimental.pallas import tpu_sc as plsc

NN = 10000
NP = 10240
DIN = 128
DD1 = 128
DD2 = 64
EP = 163840
CH = 128
NW = 32
CPW = EP // (NW * CH)
ROWS_PER_SUB = NP // 16
NBUF_A = 2
NBUF_D = 4
SLAB = 80
CF0 = 64
CF1 = 16
_F32 = jnp.float32
_DN = (((1,), (1,)), ((), ()))


def _zero_chunks():
    out, off = [], 0
    while off < ROWS_PER_SUB:
        sz = min(CH, ROWS_PER_SUB - off)
        out.append((off, sz))
        off += sz
    return out


def _make_sc_agg(SW):
    D = 128
    mesh = plsc.VectorSubcoreMesh(core_axis_name="c", subcore_axis_name="s")

    @functools.partial(
        pl.kernel,
        out_type=jax.ShapeDtypeStruct((2, NP, D), _F32),
        mesh=mesh,
        scratch_types=[
            pltpu.VMEM((CF0, CH), jnp.int32),
        ] + [pltpu.VMEM((CH,), _F32)] * NBUF_A
          + [pltpu.VMEM((CH,), jnp.int32)] * NBUF_A
          + [pltpu.VMEM((CH, D), _F32)] * NBUF_A
          + [pltpu.VMEM_SHARED((NP, D), _F32)]
          + [pltpu.SemaphoreType.DMA] * NBUF_A,
    )
    def k(feat, src2, dst2, ea1, out, src_v, ea0, ea1b, db0, db1,
          rb0, rb1, acc, sm0, sm1):
        bufs = [rb0, rb1]
        eabs = [ea0, ea1b]
        dbs = [db0, db1]
        sems = [sm0, sm1]
        c = lax.axis_index("c")
        s = lax.axis_index("s")
        base_c = pl.multiple_of(s * SLAB + c * CF0, 8)
        cnum = jnp.where(c == 0, CF0, CF1)

        def zrow(i, _):
            for kk in range(D // 16):
                rb0[i, pl.ds(16 * kk, 16)] = jnp.zeros((16,), _F32)
            return 0

        lax.fori_loop(0, CH, zrow, 0)
        for off, sz in _zero_chunks():
            base = pl.multiple_of(s * ROWS_PER_SUB + off, CH)
            pltpu.sync_copy(rb0.at[pl.ds(0, sz)], acc.at[pl.ds(base, sz)])

        @pl.when(c == 0)
        def _():
            pltpu.sync_copy(src2.at[pl.ds(base_c, CF0)], src_v)

        @pl.when(c == 1)
        def _():
            pltpu.sync_copy(src2.at[pl.ds(base_c, CF1)],
                            src_v.at[pl.ds(0, CF1)])

        plsc.subcore_barrier()

        def wait_rows(b):
            pltpu.make_async_copy(
                feat.at[pl.ds(0, CH)], bufs[b], sems[b]).wait()

        def wait_small(b):
            pltpu.make_async_copy(ea1.at[pl.ds(0, CH)], eabs[b], sems[b]).wait()
            pltpu.make_async_copy(src2.at[0], dbs[b], sems[b]).wait()

        def pipe(t, _):
            for b in range(NBUF_A):
                j = t + b
                @pl.when(j < cnum)
                def _():
                    @pl.when(j >= NBUF_A)
                    def _():
                        wait_rows(b)
                    cidx = base_c + j
                    pltpu.async_copy(ea1.at[pl.ds(cidx * CH, CH)], eabs[b],
                                     sems[b])
                    pltpu.async_copy(dst2.at[cidx], dbs[b], sems[b])
                    pltpu.async_copy(feat.at[src_v.at[j]], bufs[b], sems[b])

                jj = j - 1
                bb = (b - 1) % NBUF_A

                @pl.when((jj >= 0) & (jj < cnum))
                def _():
                    wait_rows(bb)
                    wait_small(bb)

                    def scale(gg, _):
                        eav16 = eabs[bb][pl.ds(gg * 16, 16)]
                        for i in range(16):
                            row = gg * 16 + i
                            splat = jnp.full((16,), eav16[i])
                            for kk in range(SW // 16):
                                sl = pl.ds(16 * kk, 16)
                                bufs[bb][row, sl] = bufs[bb][row, sl] * splat
                        return 0

                    lax.fori_loop(0, CH // 16, scale, 0)
                    pltpu.async_copy(bufs[bb], acc.at[dbs[bb]], sems[bb],
                                     add=True)
            return 0

        lax.fori_loop(0, (CF0 + NBUF_A) // NBUF_A,
                      lambda t, u: pipe(t * NBUF_A, u), 0, unroll=False)
        for b in range(NBUF_A):
            wait_rows(b)
        plsc.subcore_barrier()
        for off, sz in _zero_chunks():
            base = pl.multiple_of(s * ROWS_PER_SUB + off, CH)
            pltpu.sync_copy(acc.at[pl.ds(base, sz)], out.at[c, pl.ds(base, sz)])

    return k


_sc_agg128 = _make_sc_agg(DIN)
_sc_agg64 = _make_sc_agg(DD2)


def _make_sc_dec():
    mesh = plsc.VectorSubcoreMesh(core_axis_name="c", subcore_axis_name="s")

    @functools.partial(
        pl.kernel,
        out_type=jax.ShapeDtypeStruct((EP, 2 * DD2), _F32),
        mesh=mesh,
        scratch_types=[
            pltpu.VMEM((CF0, CH), jnp.int32),
            pltpu.VMEM((CF0, CH), jnp.int32),
        ] + [pltpu.VMEM((CH, 2 * DD2), _F32)] * NBUF_D
          + [pltpu.SemaphoreType.DMA] * NBUF_D,
    )
    def k(av, bv, src2, dst2, g, src_v, dst_v,
          rb0, rb1, rb2, rb3, sm0, sm1, sm2, sm3):
        bufs = [rb0, rb1, rb2, rb3]
        sems = [sm0, sm1, sm2, sm3]
        c = lax.axis_index("c")
        s = lax.axis_index("s")
        base_c = pl.multiple_of(s * SLAB + c * CF0, 8)
        cnum = jnp.where(c == 0, CF0, CF1)

        @pl.when(c == 0)
        def _():
            pltpu.sync_copy(src2.at[pl.ds(base_c, CF0)], src_v)
            pltpu.sync_copy(dst2.at[pl.ds(base_c, CF0)], dst_v)

        @pl.when(c == 1)
        def _():
            pltpu.sync_copy(src2.at[pl.ds(base_c, CF1)],
                            src_v.at[pl.ds(0, CF1)])
            pltpu.sync_copy(dst2.at[pl.ds(base_c, CF1)],
                            dst_v.at[pl.ds(0, CF1)])

        def wait(b):
            pltpu.make_async_copy(
                av.at[pl.ds(0, CH)], bufs[b], sems[b]).wait()

        def pipe(t, _):
            for b in range(NBUF_D):
                j = t + b
                @pl.when(j < cnum)
                def _():
                    @pl.when(j >= NBUF_D)
                    def _():
                        wait(b)
                    pltpu.async_copy(av.at[src_v.at[j]], bufs[b], sems[b])

                j1 = j - 1
                bb1 = (b - 1) % NBUF_D

                @pl.when((j1 >= 0) & (j1 < cnum))
                def _():
                    wait(bb1)
                    pltpu.async_copy(bv.at[dst_v.at[j1]], bufs[bb1], sems[bb1],
                                     add=True)

                j2 = j - 2
                bb2 = (b - 2) % NBUF_D

                @pl.when((j2 >= 0) & (j2 < cnum))
                def _():
                    wait(bb2)
                    pltpu.async_copy(
                        bufs[bb2], g.at[pl.ds((base_c + j2) * CH, CH)],
                        sems[bb2])
            return 0

        lax.fori_loop(0, (CF0 + NBUF_D) // NBUF_D,
                      lambda t, u: pipe(t * NBUF_D, u), 0, unroll=False)
        for b in range(NBUF_D):
            wait(b)

    return k


_sc_dec = _make_sc_dec()


def _tc1_body(aggp, xr, w1r, w1o, b1r, w2r, w2o, b2r, hw, hroot):
    agg = aggp[0] + aggp[1]
    h = lax.dot_general(agg, w1r[...], _DN, preferred_element_type=_F32)
    h = h + lax.dot_general(xr[...], w1o[...], _DN, preferred_element_type=_F32)
    h = jnp.maximum(h + b1r[...][None, :], 0.0)
    hwv = lax.dot_general(h, w2r[...], _DN, preferred_element_type=_F32)
    hw[...] = jnp.concatenate(
        [hwv, jnp.zeros((hwv.shape[0], DIN - DD2), _F32)], axis=1)
    hroot[...] = (lax.dot_general(h, w2o[...], _DN, preferred_element_type=_F32)
                  + b2r[...][None, :])


_TC1_BN = 1000


def _tc1(aggp, x, w1r, w1o, b1, w2r, w2o, b2):
    grid = (NN // _TC1_BN,)
    return pl.pallas_call(
        _tc1_body,
        grid=grid,
        in_specs=[
            pl.BlockSpec((2, _TC1_BN, DIN), lambda i: (0, i, 0)),
            pl.BlockSpec((_TC1_BN, DIN), lambda i: (i, 0)),
            pl.BlockSpec((DD1, DIN), lambda i: (0, 0)),
            pl.BlockSpec((DD1, DIN), lambda i: (0, 0)),
            pl.BlockSpec((DD1,), lambda i: (0,)),
            pl.BlockSpec((DD2, DD1), lambda i: (0, 0)),
            pl.BlockSpec((DD2, DD1), lambda i: (0, 0)),
            pl.BlockSpec((DD2,), lambda i: (0,)),
        ],
        out_specs=[
            pl.BlockSpec((_TC1_BN, DIN), lambda i: (i, 0)),
            pl.BlockSpec((_TC1_BN, DD2), lambda i: (i, 0)),
        ],
        out_shape=[
            jax.ShapeDtypeStruct((NN, DIN), _F32),
            jax.ShapeDtypeStruct((NN, DD2), _F32),
        ],
    )(aggp, x, w1r, w1o, b1, w2r, w2o, b2)


def _tc2_body(aggp, hroot, fcw, fcb, z2):
    zr = jnp.maximum(aggp[0] + aggp[1] + hroot[...], 0.0)
    p = jnp.sum(fcw[...] * zr[None, :], axis=1)

    @pl.when(pl.program_id(0) == 0)
    def _():
        z2[...] = p + fcb[...]

    @pl.when(pl.program_id(0) != 0)
    def _():
        z2[...] = z2[...] + p


_TC2_BC = 25600


def _tc2(aggp_f, hroot_f, fc_W, fc_b):
    grid = (NN * DD2 // _TC2_BC,)
    return pl.pallas_call(
        _tc2_body,
        grid=grid,
        in_specs=[
            pl.BlockSpec((2, _TC2_BC), lambda i: (0, i)),
            pl.BlockSpec((_TC2_BC,), lambda i: (i,)),
            pl.BlockSpec((DD2, _TC2_BC), lambda i: (0, i)),
            pl.BlockSpec((DD2,), lambda i: (0,)),
        ],
        out_specs=pl.BlockSpec((DD2,), lambda i: (0,)),
        out_shape=jax.ShapeDtypeStruct((DD2,), _F32),
    )(aggp_f, hroot_f, fc_W, fc_b)


def _tc3a_body(w2d, fcb2, z2, zzf):
    t = jnp.sum(w2d[...] * z2[...][None, :], axis=1) + fcb2[...]
    zzf[...] = jnp.maximum(t, 0.0)


_TC3A_BR = 5120


def _tc3a(fc2_W, fc2_b, z2):
    grid = (NN * DD2 // _TC3A_BR,)
    return pl.pallas_call(
        _tc3a_body,
        grid=grid,
        in_specs=[
            pl.BlockSpec((_TC3A_BR, DD2), lambda i: (i, 0)),
            pl.BlockSpec((_TC3A_BR,), lambda i: (i,)),
            pl.BlockSpec((DD2,), lambda i: (0,)),
        ],
        out_specs=pl.BlockSpec((_TC3A_BR,), lambda i: (i,)),
        out_shape=jax.ShapeDtypeStruct((NN * DD2,), _F32),
    )(fc2_W, fc2_b, z2)


def _tc3b_body(zz, dwa, dwb, db1, av, bv):
    z = zz[...]
    av[...] = (lax.dot_general(z, dwa[...], _DN, preferred_element_type=_F32)
               + db1[...][None, :])
    bv[...] = lax.dot_general(z, dwb[...], _DN, preferred_element_type=_F32)


_TC3B_BN = 1000


def _tc3b(zz2d, dwa, dwb, db1):
    grid = (NN // _TC3B_BN,)
    return pl.pallas_call(
        _tc3b_body,
        grid=grid,
        in_specs=[
            pl.BlockSpec((_TC3B_BN, DD2), lambda i: (i, 0)),
            pl.BlockSpec((DD1, DD2), lambda i: (0, 0)),
            pl.BlockSpec((DD1, DD2), lambda i: (0, 0)),
            pl.BlockSpec((DD1,), lambda i: (0,)),
        ],
        out_specs=[
            pl.BlockSpec((_TC3B_BN, DD1), lambda i: (i, 0)),
            pl.BlockSpec((_TC3B_BN, DD1), lambda i: (i, 0)),
        ],
        out_shape=[
            jax.ShapeDtypeStruct((NN, DD1), _F32),
            jax.ShapeDtypeStruct((NN, DD1), _F32),
        ],
    )(zz2d, dwa, dwb, db1)


def _tc4_body(g, w2, b2, out):
    t = jnp.sum(jnp.maximum(g[...], 0.0) * w2[...][0][None, :], axis=1) + b2[0]
    out[...] = 1.0 / (1.0 + jnp.exp(-t))


_TC4_BE = 2048


def _tc4(g, dec_W2, dec_b2):
    grid = (EP // _TC4_BE,)
    return pl.pallas_call(
        _tc4_body,
        grid=grid,
        in_specs=[
            pl.BlockSpec((_TC4_BE, DD1), lambda i: (i, 0)),
            pl.BlockSpec((1, DD1), lambda i: (0, 0)),
            pl.BlockSpec(memory_space=pltpu.SMEM),
        ],
        out_specs=pl.BlockSpec((_TC4_BE,), lambda i: (i,)),
        out_shape=jax.ShapeDtypeStruct((EP,), _F32),
    )(g, dec_W2, dec_b2)


def kernel(x, edge_index, edge_attr, W1_rel, b1, W1_root, W2_rel, b2, W2_root,
           fc_W, fc_b, fc2_W, fc2_b, dec_W1, dec_b1, dec_W2, dec_b2):
    E = edge_index.shape[1]
    pad = EP - E
    src_p = jnp.concatenate(
        [edge_index[0], jnp.zeros((pad,), jnp.int32)]).reshape(EP // CH, CH)
    dst_p = jnp.concatenate(
        [edge_index[1], jnp.zeros((pad,), jnp.int32)]).reshape(EP // CH, CH)
    ea_p = jnp.concatenate([edge_attr, jnp.zeros((pad,), _F32)])

    aggp = _sc_agg128(x, src_p, dst_p, ea_p)
    hw, hroot = _tc1(aggp, x, W1_rel, W1_root, b1, W2_rel, W2_root, b2)
    agg2p = _sc_agg64(hw, src_p, dst_p, ea_p)
    a2f = agg2p[:, :NN, :DD2].reshape(2, NN * DD2)
    z2 = _tc2(a2f, hroot.reshape(NN * DD2), fc_W, fc_b)
    zzf = _tc3a(fc2_W, fc2_b, z2)
    av, bv = _tc3b(zzf.reshape(NN, DD2), dec_W1[:, :DD2], dec_W1[:, DD2:],
                   dec_b1)
    g = _sc_dec(av, bv, src_p, dst_p)
    outp = _tc4(g, dec_W2, dec_b2)
    return outp[:E]


# speedup vs baseline: 2.5488x; 1.4888x over previous
import functools

import jax
import jax.numpy as jnp
from jax import lax
from jax.experimental import pallas as pl
from jax.experimental.pallas import tpu as pltpu
from jax.experimental.pallas import tpu_sc as plsc

NN = 10000
NP = 10240
DIN = 128
DD1 = 128
DD2 = 64
EP = 163840
CH = 128
NW = 32
CPW = EP // (NW * CH)
ROWS_PER_SUB = NP // 16
NBUF_A = 2
NBUF_D = 4
SLAB = 80
CF0 = 64
CF1 = 16
_F32 = jnp.float32
_DN = (((1,), (1,)), ((), ()))
_HI = jax.lax.Precision.HIGHEST


def _zero_chunks():
    out, off = [], 0
    while off < ROWS_PER_SUB:
        sz = min(CH, ROWS_PER_SUB - off)
        out.append((off, sz))
        off += sz
    return out


def _make_sc_agg(SW):
    D = 128
    mesh = plsc.VectorSubcoreMesh(core_axis_name="c", subcore_axis_name="s")

    @functools.partial(
        pl.kernel,
        out_type=jax.ShapeDtypeStruct((2, NP, D), _F32),
        mesh=mesh,
        scratch_types=[
            pltpu.VMEM((CF0, CH), jnp.int32),
        ] + [pltpu.VMEM((CH,), _F32)] * NBUF_A
          + [pltpu.VMEM((CH,), jnp.int32)] * NBUF_A
          + [pltpu.VMEM((CH, D), _F32)] * NBUF_A
          + [pltpu.VMEM_SHARED((NP, D), _F32)]
          + [pltpu.SemaphoreType.DMA] * NBUF_A,
    )
    def k(feat, src2, dst2, ea1, out, src_v, ea0, ea1b, db0, db1,
          rb0, rb1, acc, sm0, sm1):
        bufs = [rb0, rb1]
        eabs = [ea0, ea1b]
        dbs = [db0, db1]
        sems = [sm0, sm1]
        c = lax.axis_index("c")
        s = lax.axis_index("s")
        base_c = pl.multiple_of(s * SLAB + c * CF0, 8)
        cnum = jnp.where(c == 0, CF0, CF1)

        def zrow(i, _):
            for kk in range(D // 16):
                rb0[i, pl.ds(16 * kk, 16)] = jnp.zeros((16,), _F32)
            return 0

        lax.fori_loop(0, CH, zrow, 0)
        for off, sz in _zero_chunks():
            base = pl.multiple_of(s * ROWS_PER_SUB + off, CH)
            pltpu.sync_copy(rb0.at[pl.ds(0, sz)], acc.at[pl.ds(base, sz)])

        @pl.when(c == 0)
        def _():
            pltpu.sync_copy(src2.at[pl.ds(base_c, CF0)], src_v)

        @pl.when(c == 1)
        def _():
            pltpu.sync_copy(src2.at[pl.ds(base_c, CF1)],
                            src_v.at[pl.ds(0, CF1)])

        plsc.subcore_barrier()

        def wait_rows(b):
            pltpu.make_async_copy(
                feat.at[pl.ds(0, CH)], bufs[b], sems[b]).wait()

        def wait_small(b):
            pltpu.make_async_copy(ea1.at[pl.ds(0, CH)], eabs[b], sems[b]).wait()
            pltpu.make_async_copy(src2.at[0], dbs[b], sems[b]).wait()

        def pipe(t, _):
            for b in range(NBUF_A):
                j = t + b
                @pl.when(j < cnum)
                def _():
                    @pl.when(j >= NBUF_A)
                    def _():
                        wait_rows(b)
                    cidx = base_c + j
                    pltpu.async_copy(ea1.at[pl.ds(cidx * CH, CH)], eabs[b],
                                     sems[b])
                    pltpu.async_copy(dst2.at[cidx], dbs[b], sems[b])
                    pltpu.async_copy(feat.at[src_v.at[j]], bufs[b], sems[b])

                jj = j - 1
                bb = (b - 1) % NBUF_A

                @pl.when((jj >= 0) & (jj < cnum))
                def _():
                    wait_rows(bb)
                    wait_small(bb)

                    def scale(gg, _):
                        eav16 = eabs[bb][pl.ds(gg * 16, 16)]
                        for i in range(16):
                            row = gg * 16 + i
                            splat = jnp.full((16,), eav16[i])
                            for kk in range(SW // 16):
                                sl = pl.ds(16 * kk, 16)
                                bufs[bb][row, sl] = bufs[bb][row, sl] * splat
                        return 0

                    lax.fori_loop(0, CH // 16, scale, 0)
                    pltpu.async_copy(bufs[bb], acc.at[dbs[bb]], sems[bb],
                                     add=True)
            return 0

        lax.fori_loop(0, (CF0 + NBUF_A) // NBUF_A,
                      lambda t, u: pipe(t * NBUF_A, u), 0, unroll=False)
        for b in range(NBUF_A):
            wait_rows(b)
        plsc.subcore_barrier()
        for off, sz in _zero_chunks():
            base = pl.multiple_of(s * ROWS_PER_SUB + off, CH)
            pltpu.sync_copy(acc.at[pl.ds(base, sz)], out.at[c, pl.ds(base, sz)])

    return k


_sc_agg128 = _make_sc_agg(DIN)
_sc_agg64 = _make_sc_agg(DD2)


def _make_sc_dec():
    mesh = plsc.VectorSubcoreMesh(core_axis_name="c", subcore_axis_name="s")

    @functools.partial(
        pl.kernel,
        out_type=jax.ShapeDtypeStruct((EP // 8, 2 * DD2), _F32),
        mesh=mesh,
        scratch_types=[
            pltpu.VMEM((CF0, CH), jnp.int32),
            pltpu.VMEM((CF0, CH), jnp.int32),
            pltpu.VMEM((2 * DD2,), _F32),
        ] + [pltpu.VMEM((CH, 2 * DD2), _F32)] * NBUF_D
          + [pltpu.VMEM((CH // 8, 2 * DD2), _F32)]
          + [pltpu.SemaphoreType.DMA] * (NBUF_D + 1),
    )
    def k(av, bv, w2h, src2, dst2, g, src_v, dst_v, w2v,
          rb0, rb1, rb2, rb3, pb0, sm0, sm1, sm2, sm3, psm):
        bufs = [rb0, rb1, rb2, rb3]
        sems = [sm0, sm1, sm2, sm3]
        c = lax.axis_index("c")
        s = lax.axis_index("s")
        base_c = pl.multiple_of(s * SLAB + c * CF0, 8)
        cnum = jnp.where(c == 0, CF0, CF1)
        pltpu.sync_copy(w2h.at[0], w2v)

        @pl.when(c == 0)
        def _():
            pltpu.sync_copy(src2.at[pl.ds(base_c, CF0)], src_v)
            pltpu.sync_copy(dst2.at[pl.ds(base_c, CF0)], dst_v)

        @pl.when(c == 1)
        def _():
            pltpu.sync_copy(src2.at[pl.ds(base_c, CF1)],
                            src_v.at[pl.ds(0, CF1)])
            pltpu.sync_copy(dst2.at[pl.ds(base_c, CF1)],
                            dst_v.at[pl.ds(0, CF1)])

        def wait(b):
            pltpu.make_async_copy(
                av.at[pl.ds(0, CH)], bufs[b], sems[b]).wait()

        def pwait():
            pltpu.make_async_copy(
                av.at[pl.ds(0, CH // 8)], pb0, psm).wait()

        def pipe(t, _):
            for b in range(NBUF_D):
                j = t + b
                @pl.when(j < cnum)
                def _():
                    pltpu.async_copy(av.at[src_v.at[j]], bufs[b], sems[b])

                j1 = j - 1
                bb1 = (b - 1) % NBUF_D

                @pl.when((j1 >= 0) & (j1 < cnum))
                def _():
                    wait(bb1)
                    pltpu.async_copy(bv.at[dst_v.at[j1]], bufs[bb1], sems[bb1],
                                     add=True)

                j2 = j - 2
                bb2 = (b - 2) % NBUF_D

                @pl.when((j2 >= 0) & (j2 < cnum))
                def _():
                    wait(bb2)

                    @pl.when(j2 >= 1)
                    def _():
                        pwait()

                    def red(gq, _):
                        for q in range(8):
                            row = gq * 8 + q
                            acc = jnp.zeros((16,), _F32)
                            for kk in range(8):
                                sl = pl.ds(16 * kk, 16)
                                acc = acc + (jnp.maximum(bufs[bb2][row, sl],
                                                         0.0) * w2v[sl])
                            pb0[gq, pl.ds(16 * q, 16)] = acc
                        return 0

                    lax.fori_loop(0, CH // 8, red, 0)
                    pltpu.async_copy(
                        pb0, g.at[pl.ds((base_c + j2) * (CH // 8), CH // 8)],
                        psm)
            return 0

        lax.fori_loop(0, (CF0 + NBUF_D) // NBUF_D,
                      lambda t, u: pipe(t * NBUF_D, u), 0, unroll=False)
        pwait()

    return k


_sc_dec = _make_sc_dec()


def _tc1_body(aggp, xr, w1r, w1o, b1r, w2r, w2o, b2r, hw, hroot):
    agg = aggp[0] + aggp[1]
    h = lax.dot_general(agg, w1r[...], _DN, precision=_HI, preferred_element_type=_F32)
    h = h + lax.dot_general(xr[...], w1o[...], _DN, precision=_HI, preferred_element_type=_F32)
    h = jnp.maximum(h + b1r[...][None, :], 0.0)
    hwv = lax.dot_general(h, w2r[...], _DN, precision=_HI, preferred_element_type=_F32)
    hw[...] = jnp.concatenate(
        [hwv, jnp.zeros((hwv.shape[0], DIN - DD2), _F32)], axis=1)
    hroot[...] = (lax.dot_general(h, w2o[...], _DN, precision=_HI, preferred_element_type=_F32)
                  + b2r[...][None, :])


_TC1_BN = 1000


def _tc1(aggp, x, w1r, w1o, b1, w2r, w2o, b2):
    grid = (NN // _TC1_BN,)
    return pl.pallas_call(
        _tc1_body,
        grid=grid,
        in_specs=[
            pl.BlockSpec((2, _TC1_BN, DIN), lambda i: (0, i, 0)),
            pl.BlockSpec((_TC1_BN, DIN), lambda i: (i, 0)),
            pl.BlockSpec((DD1, DIN), lambda i: (0, 0)),
            pl.BlockSpec((DD1, DIN), lambda i: (0, 0)),
            pl.BlockSpec((DD1,), lambda i: (0,)),
            pl.BlockSpec((DD2, DD1), lambda i: (0, 0)),
            pl.BlockSpec((DD2, DD1), lambda i: (0, 0)),
            pl.BlockSpec((DD2,), lambda i: (0,)),
        ],
        out_specs=[
            pl.BlockSpec((_TC1_BN, DIN), lambda i: (i, 0)),
            pl.BlockSpec((_TC1_BN, DD2), lambda i: (i, 0)),
        ],
        out_shape=[
            jax.ShapeDtypeStruct((NN, DIN), _F32),
            jax.ShapeDtypeStruct((NN, DD2), _F32),
        ],
    )(aggp, x, w1r, w1o, b1, w2r, w2o, b2)


def _tc2_body(aggp, hroot, fcw, fcb, z2):
    zr = jnp.maximum(aggp[0] + aggp[1] + hroot[...], 0.0)
    p = jnp.sum(fcw[...] * zr[None, :], axis=1)

    @pl.when(pl.program_id(0) == 0)
    def _():
        z2[...] = p + fcb[...]

    @pl.when(pl.program_id(0) != 0)
    def _():
        z2[...] = z2[...] + p


_TC2_BC = 25600


def _tc2(aggp_f, hroot_f, fc_W, fc_b):
    grid = (NN * DD2 // _TC2_BC,)
    return pl.pallas_call(
        _tc2_body,
        grid=grid,
        in_specs=[
            pl.BlockSpec((2, _TC2_BC), lambda i: (0, i)),
            pl.BlockSpec((_TC2_BC,), lambda i: (i,)),
            pl.BlockSpec((DD2, _TC2_BC), lambda i: (0, i)),
            pl.BlockSpec((DD2,), lambda i: (0,)),
        ],
        out_specs=pl.BlockSpec((DD2,), lambda i: (0,)),
        out_shape=jax.ShapeDtypeStruct((DD2,), _F32),
    )(aggp_f, hroot_f, fc_W, fc_b)


def _tc3a_body(w2d, fcb2, z2, zzf):
    t = jnp.sum(w2d[...] * z2[...][:, None], axis=0) + fcb2[...]
    zzf[...] = jnp.maximum(t, 0.0)


_TC3A_BR = 25600


def _tc3a(fc2_W_T, fc2_b, z2):
    grid = (NN * DD2 // _TC3A_BR,)
    return pl.pallas_call(
        _tc3a_body,
        grid=grid,
        in_specs=[
            pl.BlockSpec((DD2, _TC3A_BR), lambda i: (0, i)),
            pl.BlockSpec((_TC3A_BR,), lambda i: (i,)),
            pl.BlockSpec((DD2,), lambda i: (0,)),
        ],
        out_specs=pl.BlockSpec((_TC3A_BR,), lambda i: (i,)),
        out_shape=jax.ShapeDtypeStruct((NN * DD2,), _F32),
    )(fc2_W_T, fc2_b, z2)


def _tc3b_body(zz, dwa, dwb, db1, av, bv):
    z = zz[...]
    av[...] = (lax.dot_general(z, dwa[...], _DN, precision=_HI, preferred_element_type=_F32)
               + db1[...][None, :])
    bv[...] = lax.dot_general(z, dwb[...], _DN, precision=_HI, preferred_element_type=_F32)


_TC3B_BN = 1000


def _tc3b(zz2d, dwa, dwb, db1):
    grid = (NN // _TC3B_BN,)
    return pl.pallas_call(
        _tc3b_body,
        grid=grid,
        in_specs=[
            pl.BlockSpec((_TC3B_BN, DD2), lambda i: (i, 0)),
            pl.BlockSpec((DD1, DD2), lambda i: (0, 0)),
            pl.BlockSpec((DD1, DD2), lambda i: (0, 0)),
            pl.BlockSpec((DD1,), lambda i: (0,)),
        ],
        out_specs=[
            pl.BlockSpec((_TC3B_BN, DD1), lambda i: (i, 0)),
            pl.BlockSpec((_TC3B_BN, DD1), lambda i: (i, 0)),
        ],
        out_shape=[
            jax.ShapeDtypeStruct((NN, DD1), _F32),
            jax.ShapeDtypeStruct((NN, DD1), _F32),
        ],
    )(zz2d, dwa, dwb, db1)


def _tc4_body(g, b2, *outs):
    x = g[...]
    for q in range(8):
        t = jnp.sum(x[:, 16 * q:16 * (q + 1)], axis=1) + b2[0]
        outs[q][...] = 1.0 / (1.0 + jnp.exp(-t))


_TC4_BE = 2048


def _tc4(g2, dec_b2):
    grid = (EP // 8 // _TC4_BE,)
    return pl.pallas_call(
        _tc4_body,
        grid=grid,
        in_specs=[
            pl.BlockSpec((_TC4_BE, 2 * DD2), lambda i: (i, 0)),
            pl.BlockSpec(memory_space=pltpu.SMEM),
        ],
        out_specs=[pl.BlockSpec((_TC4_BE,), lambda i: (i,))] * 8,
        out_shape=[jax.ShapeDtypeStruct((EP // 8,), _F32)] * 8,
    )(g2, dec_b2)


def kernel(x, edge_index, edge_attr, W1_rel, b1, W1_root, W2_rel, b2, W2_root,
           fc_W, fc_b, fc2_W, fc2_b, dec_W1, dec_b1, dec_W2, dec_b2):
    E = edge_index.shape[1]
    pad = EP - E
    src_p = jnp.concatenate(
        [edge_index[0], jnp.zeros((pad,), jnp.int32)]).reshape(EP // CH, CH)
    dst_p = jnp.concatenate(
        [edge_index[1], jnp.zeros((pad,), jnp.int32)]).reshape(EP // CH, CH)
    ea_p = jnp.concatenate([edge_attr, jnp.zeros((pad,), _F32)])

    aggp = _sc_agg128(x, src_p, dst_p, ea_p)
    hw, hroot = _tc1(aggp, x, W1_rel, W1_root, b1, W2_rel, W2_root, b2)
    agg2p = _sc_agg64(hw, src_p, dst_p, ea_p)
    a2f = agg2p[:, :NN, :DD2].reshape(2, NN * DD2)
    z2 = _tc2(a2f, hroot.reshape(NN * DD2), fc_W, fc_b)
    zzf = _tc3a(fc2_W.T, fc2_b, z2)
    av, bv = _tc3b(zzf.reshape(NN, DD2), dec_W1[:, :DD2], dec_W1[:, DD2:],
                   dec_b1)
    g2 = _sc_dec(av, bv, dec_W2, src_p, dst_p)
    outq = _tc4(g2, dec_b2)
    outp = jnp.stack(outq, axis=1).reshape(EP)
    return outp[:E]


# speedup vs baseline: 2.7429x; 1.0761x over previous
import functools

import jax
import jax.numpy as jnp
from jax import lax
from jax.experimental import pallas as pl
from jax.experimental.pallas import tpu as pltpu
from jax.experimental.pallas import tpu_sc as plsc

NN = 10000
NP = 10240
DIN = 128
DD1 = 128
DD2 = 64
EP = 163840
CH = 128
NW = 32
CPW = EP // (NW * CH)
ROWS_PER_SUB = NP // 16
NBUF_A = 3
CHA = 64
SLABA = 160
CF0A = 128
CF1A = 32
NBUF_D = 4
SLAB = 80
CF0 = 64
CF1 = 16
_F32 = jnp.float32
_DN = (((1,), (1,)), ((), ()))
_HI = jax.lax.Precision.HIGHEST


def _zero_chunks(step):
    out, off = [], 0
    while off < ROWS_PER_SUB:
        sz = min(step, ROWS_PER_SUB - off)
        out.append((off, sz))
        off += sz
    return out


def _make_sc_agg(SW):
    D = 128
    mesh = plsc.VectorSubcoreMesh(core_axis_name="c", subcore_axis_name="s")

    @functools.partial(
        pl.kernel,
        out_type=jax.ShapeDtypeStruct((2, NP, D), _F32),
        mesh=mesh,
        scratch_types=[
            pltpu.VMEM((CF0A, CHA), jnp.int32),
        ] + [pltpu.VMEM((CHA,), _F32)] * NBUF_A
          + [pltpu.VMEM((CHA,), jnp.int32)] * NBUF_A
          + [pltpu.VMEM((CHA, D), _F32)] * NBUF_A
          + [pltpu.VMEM_SHARED((NP, D), _F32)]
          + [pltpu.SemaphoreType.DMA] * NBUF_A,
    )
    def k(feat, src2, dst2, ea1, out, src_v, ea0, ea1b, ea2b,
          db0, db1, db2, rb0, rb1, rb2, acc, sm0, sm1, sm2):
        bufs = [rb0, rb1, rb2]
        eabs = [ea0, ea1b, ea2b]
        dbs = [db0, db1, db2]
        sems = [sm0, sm1, sm2]
        c = lax.axis_index("c")
        s = lax.axis_index("s")
        base_c = pl.multiple_of(s * SLABA + c * CF0A, 8)
        cnum = jnp.where(c == 0, CF0A, CF1A)

        def zrow(i, _):
            for kk in range(D // 16):
                rb0[i, pl.ds(16 * kk, 16)] = jnp.zeros((16,), _F32)
            return 0

        lax.fori_loop(0, CHA, zrow, 0)
        for off, sz in _zero_chunks(CHA):
            base = pl.multiple_of(s * ROWS_PER_SUB + off, CHA)
            pltpu.sync_copy(rb0.at[pl.ds(0, sz)], acc.at[pl.ds(base, sz)])

        @pl.when(c == 0)
        def _():
            pltpu.sync_copy(src2.at[pl.ds(base_c, CF0A)], src_v)

        @pl.when(c == 1)
        def _():
            pltpu.sync_copy(src2.at[pl.ds(base_c, CF1A)],
                            src_v.at[pl.ds(0, CF1A)])

        plsc.subcore_barrier()

        def wait_rows(b):
            pltpu.make_async_copy(
                feat.at[pl.ds(0, CHA)], bufs[b], sems[b]).wait()

        def wait_small(b):
            pltpu.make_async_copy(ea1.at[pl.ds(0, CHA)], eabs[b],
                                  sems[b]).wait()
            pltpu.make_async_copy(src2.at[0], dbs[b], sems[b]).wait()

        def pipe(t, _):
            for b in range(NBUF_A):
                j = t + b
                @pl.when(j < cnum)
                def _():
                    @pl.when(j >= NBUF_A)
                    def _():
                        wait_rows(b)
                    cidx = base_c + j
                    pltpu.async_copy(ea1.at[pl.ds(cidx * CHA, CHA)], eabs[b],
                                     sems[b])
                    pltpu.async_copy(dst2.at[cidx], dbs[b], sems[b])
                    pltpu.async_copy(feat.at[src_v.at[j]], bufs[b], sems[b])

                jj = j - 2
                bb = (b - 2) % NBUF_A

                @pl.when((jj >= 0) & (jj < cnum))
                def _():
                    wait_rows(bb)
                    wait_small(bb)

                    def scale(gg, _):
                        eav16 = eabs[bb][pl.ds(gg * 16, 16)]
                        for i in range(16):
                            row = gg * 16 + i
                            splat = jnp.full((16,), eav16[i])
                            for kk in range(SW // 16):
                                sl = pl.ds(16 * kk, 16)
                                bufs[bb][row, sl] = bufs[bb][row, sl] * splat
                        return 0

                    lax.fori_loop(0, CHA // 16, scale, 0)
                    pltpu.async_copy(bufs[bb], acc.at[dbs[bb]], sems[bb],
                                     add=True)
            return 0

        lax.fori_loop(0, (CF0A + 2 * NBUF_A) // NBUF_A,
                      lambda t, u: pipe(t * NBUF_A, u), 0, unroll=False)
        for b in range(NBUF_A):
            wait_rows(b)
        plsc.subcore_barrier()
        for off, sz in _zero_chunks(CHA):
            base = pl.multiple_of(s * ROWS_PER_SUB + off, CHA)
            pltpu.sync_copy(acc.at[pl.ds(base, sz)], out.at[c, pl.ds(base, sz)])

    return k


_sc_agg128 = _make_sc_agg(DIN)
_sc_agg64 = _make_sc_agg(DD2)


def _make_sc_dec():
    mesh = plsc.VectorSubcoreMesh(core_axis_name="c", subcore_axis_name="s")

    @functools.partial(
        pl.kernel,
        out_type=jax.ShapeDtypeStruct((EP // 8, 2 * DD2), _F32),
        mesh=mesh,
        scratch_types=[
            pltpu.VMEM((CF0, CH), jnp.int32),
            pltpu.VMEM((CF0, CH), jnp.int32),
            pltpu.VMEM((2 * DD2,), _F32),
        ] + [pltpu.VMEM((CH, 2 * DD2), _F32)] * NBUF_D
          + [pltpu.VMEM((CH // 8, 2 * DD2), _F32)]
          + [pltpu.SemaphoreType.DMA] * (NBUF_D + 1),
    )
    def k(av, bv, w2h, src2, dst2, g, src_v, dst_v, w2v,
          rb0, rb1, rb2, rb3, pb0, sm0, sm1, sm2, sm3, psm):
        bufs = [rb0, rb1, rb2, rb3]
        sems = [sm0, sm1, sm2, sm3]
        c = lax.axis_index("c")
        s = lax.axis_index("s")
        base_c = pl.multiple_of(s * SLAB + c * CF0, 8)
        cnum = jnp.where(c == 0, CF0, CF1)
        pltpu.sync_copy(w2h.at[0], w2v)

        @pl.when(c == 0)
        def _():
            pltpu.sync_copy(src2.at[pl.ds(base_c, CF0)], src_v)
            pltpu.sync_copy(dst2.at[pl.ds(base_c, CF0)], dst_v)

        @pl.when(c == 1)
        def _():
            pltpu.sync_copy(src2.at[pl.ds(base_c, CF1)],
                            src_v.at[pl.ds(0, CF1)])
            pltpu.sync_copy(dst2.at[pl.ds(base_c, CF1)],
                            dst_v.at[pl.ds(0, CF1)])

        def wait(b):
            pltpu.make_async_copy(
                av.at[pl.ds(0, CH)], bufs[b], sems[b]).wait()

        def pwait():
            pltpu.make_async_copy(
                av.at[pl.ds(0, CH // 8)], pb0, psm).wait()

        def pipe(t, _):
            for b in range(NBUF_D):
                j = t + b
                @pl.when(j < cnum)
                def _():
                    pltpu.async_copy(av.at[src_v.at[j]], bufs[b], sems[b])

                j1 = j - 1
                bb1 = (b - 1) % NBUF_D

                @pl.when((j1 >= 0) & (j1 < cnum))
                def _():
                    wait(bb1)
                    pltpu.async_copy(bv.at[dst_v.at[j1]], bufs[bb1], sems[bb1],
                                     add=True)

                j2 = j - 2
                bb2 = (b - 2) % NBUF_D

                @pl.when((j2 >= 0) & (j2 < cnum))
                def _():
                    wait(bb2)

                    @pl.when(j2 >= 1)
                    def _():
                        pwait()

                    def red(gq, _):
                        for q in range(8):
                            row = gq * 8 + q
                            acc = jnp.zeros((16,), _F32)
                            for kk in range(8):
                                sl = pl.ds(16 * kk, 16)
                                acc = acc + (jnp.maximum(bufs[bb2][row, sl],
                                                         0.0) * w2v[sl])
                            pb0[gq, pl.ds(16 * q, 16)] = acc
                        return 0

                    lax.fori_loop(0, CH // 8, red, 0)
                    pltpu.async_copy(
                        pb0, g.at[pl.ds((base_c + j2) * (CH // 8), CH // 8)],
                        psm)
            return 0

        lax.fori_loop(0, (CF0 + NBUF_D) // NBUF_D,
                      lambda t, u: pipe(t * NBUF_D, u), 0, unroll=False)
        pwait()

    return k


_sc_dec = _make_sc_dec()


def _tc1_body(aggp, xr, w1r, w1o, b1r, w2r, w2o, b2r, hw, hroot):
    agg = aggp[0] + aggp[1]
    h = lax.dot_general(agg, w1r[...], _DN, precision=_HI, preferred_element_type=_F32)
    h = h + lax.dot_general(xr[...], w1o[...], _DN, precision=_HI, preferred_element_type=_F32)
    h = jnp.maximum(h + b1r[...][None, :], 0.0)
    hwv = lax.dot_general(h, w2r[...], _DN, precision=_HI, preferred_element_type=_F32)
    hw[...] = jnp.concatenate(
        [hwv, jnp.zeros((hwv.shape[0], DIN - DD2), _F32)], axis=1)
    hroot[...] = (lax.dot_general(h, w2o[...], _DN, precision=_HI, preferred_element_type=_F32)
                  + b2r[...][None, :])


_TC1_BN = 1000


def _tc1(aggp, x, w1r, w1o, b1, w2r, w2o, b2):
    grid = (NN // _TC1_BN,)
    return pl.pallas_call(
        _tc1_body,
        grid=grid,
        in_specs=[
            pl.BlockSpec((2, _TC1_BN, DIN), lambda i: (0, i, 0)),
            pl.BlockSpec((_TC1_BN, DIN), lambda i: (i, 0)),
            pl.BlockSpec((DD1, DIN), lambda i: (0, 0)),
            pl.BlockSpec((DD1, DIN), lambda i: (0, 0)),
            pl.BlockSpec((DD1,), lambda i: (0,)),
            pl.BlockSpec((DD2, DD1), lambda i: (0, 0)),
            pl.BlockSpec((DD2, DD1), lambda i: (0, 0)),
            pl.BlockSpec((DD2,), lambda i: (0,)),
        ],
        out_specs=[
            pl.BlockSpec((_TC1_BN, DIN), lambda i: (i, 0)),
            pl.BlockSpec((_TC1_BN, DD2), lambda i: (i, 0)),
        ],
        out_shape=[
            jax.ShapeDtypeStruct((NN, DIN), _F32),
            jax.ShapeDtypeStruct((NN, DD2), _F32),
        ],
    )(aggp, x, w1r, w1o, b1, w2r, w2o, b2)


def _tc2_body(aggp, hroot, fcw, fcb, z2):
    zr = jnp.maximum(aggp[0] + aggp[1] + hroot[...], 0.0)
    p = jnp.sum(fcw[...] * zr[None, :], axis=1)

    @pl.when(pl.program_id(0) == 0)
    def _():
        z2[...] = p + fcb[...]

    @pl.when(pl.program_id(0) != 0)
    def _():
        z2[...] = z2[...] + p


_TC2_BC = 25600


def _tc2(aggp_f, hroot_f, fc_W, fc_b):
    grid = (NN * DD2 // _TC2_BC,)
    return pl.pallas_call(
        _tc2_body,
        grid=grid,
        in_specs=[
            pl.BlockSpec((2, _TC2_BC), lambda i: (0, i)),
            pl.BlockSpec((_TC2_BC,), lambda i: (i,)),
            pl.BlockSpec((DD2, _TC2_BC), lambda i: (0, i)),
            pl.BlockSpec((DD2,), lambda i: (0,)),
        ],
        out_specs=pl.BlockSpec((DD2,), lambda i: (0,)),
        out_shape=jax.ShapeDtypeStruct((DD2,), _F32),
    )(aggp_f, hroot_f, fc_W, fc_b)


def _tc3a_body(w2d, fcb2, z2, zzf):
    t = jnp.sum(w2d[...] * z2[...][:, None], axis=0) + fcb2[...]
    zzf[...] = jnp.maximum(t, 0.0)


_TC3A_BR = 25600


def _tc3a(fc2_W_T, fc2_b, z2):
    grid = (NN * DD2 // _TC3A_BR,)
    return pl.pallas_call(
        _tc3a_body,
        grid=grid,
        in_specs=[
            pl.BlockSpec((DD2, _TC3A_BR), lambda i: (0, i)),
            pl.BlockSpec((_TC3A_BR,), lambda i: (i,)),
            pl.BlockSpec((DD2,), lambda i: (0,)),
        ],
        out_specs=pl.BlockSpec((_TC3A_BR,), lambda i: (i,)),
        out_shape=jax.ShapeDtypeStruct((NN * DD2,), _F32),
    )(fc2_W_T, fc2_b, z2)


def _tc3b_body(zz, dwa, dwb, db1, av, bv):
    z = zz[...]
    av[...] = (lax.dot_general(z, dwa[...], _DN, precision=_HI, preferred_element_type=_F32)
               + db1[...][None, :])
    bv[...] = lax.dot_general(z, dwb[...], _DN, precision=_HI, preferred_element_type=_F32)


_TC3B_BN = 1000


def _tc3b(zz2d, dwa, dwb, db1):
    grid = (NN // _TC3B_BN,)
    return pl.pallas_call(
        _tc3b_body,
        grid=grid,
        in_specs=[
            pl.BlockSpec((_TC3B_BN, DD2), lambda i: (i, 0)),
            pl.BlockSpec((DD1, DD2), lambda i: (0, 0)),
            pl.BlockSpec((DD1, DD2), lambda i: (0, 0)),
            pl.BlockSpec((DD1,), lambda i: (0,)),
        ],
        out_specs=[
            pl.BlockSpec((_TC3B_BN, DD1), lambda i: (i, 0)),
            pl.BlockSpec((_TC3B_BN, DD1), lambda i: (i, 0)),
        ],
        out_shape=[
            jax.ShapeDtypeStruct((NN, DD1), _F32),
            jax.ShapeDtypeStruct((NN, DD1), _F32),
        ],
    )(zz2d, dwa, dwb, db1)


def _tc4_body(g, b2, out):
    msk = (jax.lax.broadcasted_iota(jnp.int32, (2 * DD2, 8), 0) // 16
           == jax.lax.broadcasted_iota(jnp.int32, (2 * DD2, 8), 1))
    t = lax.dot_general(g[...], msk.astype(_F32),
                        (((1,), (0,)), ((), ())), precision=_HI,
                        preferred_element_type=_F32) + b2[0]
    out[...] = 1.0 / (1.0 + jnp.exp(-t))


_TC4_BE = 2048


def _tc4(g2, dec_b2):
    grid = (EP // 8 // _TC4_BE,)
    return pl.pallas_call(
        _tc4_body,
        grid=grid,
        in_specs=[
            pl.BlockSpec((_TC4_BE, 2 * DD2), lambda i: (i, 0)),
            pl.BlockSpec(memory_space=pltpu.SMEM),
        ],
        out_specs=pl.BlockSpec((_TC4_BE, 8), lambda i: (i, 0)),
        out_shape=jax.ShapeDtypeStruct((EP // 8, 8), _F32),
    )(g2, dec_b2)


def kernel(x, edge_index, edge_attr, W1_rel, b1, W1_root, W2_rel, b2, W2_root,
           fc_W, fc_b, fc2_W, fc2_b, dec_W1, dec_b1, dec_W2, dec_b2):
    E = edge_index.shape[1]
    pad = EP - E
    src_f = jnp.concatenate([edge_index[0], jnp.zeros((pad,), jnp.int32)])
    dst_f = jnp.concatenate([edge_index[1], jnp.zeros((pad,), jnp.int32)])
    src_p = src_f.reshape(EP // CH, CH)
    dst_p = dst_f.reshape(EP // CH, CH)
    src_a = src_f.reshape(EP // CHA, CHA)
    dst_a = dst_f.reshape(EP // CHA, CHA)
    ea_p = jnp.concatenate([edge_attr, jnp.zeros((pad,), _F32)])

    aggp = _sc_agg128(x, src_a, dst_a, ea_p)
    hw, hroot = _tc1(aggp, x, W1_rel, W1_root, b1, W2_rel, W2_root, b2)
    agg2p = _sc_agg64(hw, src_a, dst_a, ea_p)
    a2f = agg2p[:, :NN, :DD2].reshape(2, NN * DD2)
    z2 = _tc2(a2f, hroot.reshape(NN * DD2), fc_W, fc_b)
    zzf = _tc3a(fc2_W.T, fc2_b, z2)
    av, bv = _tc3b(zzf.reshape(NN, DD2), dec_W1[:, :DD2], dec_W1[:, DD2:],
                   dec_b1)
    g2 = _sc_dec(av, bv, dec_W2, src_p, dst_p)
    outp = _tc4(g2, dec_b2).reshape(EP)
    return outp[:E]
